# baseline (speedup 1.0000x reference)
import functools

import jax
import jax.numpy as jnp
from jax import lax
from jax.experimental import pallas as pl
from jax.experimental.pallas import tpu as pltpu
from jax.experimental.pallas import tpu_sc as plsc

N = 10000
E = 320000
D = 128

NC = 2
NS = 16
NW = NC * NS
EPT = E // NW
CH = 80
NCHUNK = EPT // CH
RPT = N // NS

_mesh = plsc.VectorSubcoreMesh(core_axis_name="c", subcore_axis_name="s")

_GD = lax.GatherDimensionNumbers(
    offset_dims=(), collapsed_slice_dims=(0,), start_index_map=(0,))


def _splat(v, r):
    idx = jnp.full((16, 1), r, jnp.int32)
    return lax.gather(v, idx, _GD, (1,),
                      mode=lax.GatherScatterMode.PROMISE_IN_BOUNDS)


def _sc_deg_body(dst_hbm, w_hbm, out_hbm, dst_v, w_v, zbuf, deg_sh):
    c = lax.axis_index("c")
    s = lax.axis_index("s")
    wid = s * NC + c
    pltpu.sync_copy(dst_hbm.at[wid], dst_v)
    pltpu.sync_copy(w_hbm.at[wid, 0], w_v)

    zeros = jnp.zeros((16,), jnp.float32)

    def zero_body(i, carry):
        zbuf[pl.ds(i * 16, 16)] = zeros
        return carry

    lax.fori_loop(0, 1024 // 16, zero_body, 0)

    @pl.when(s < 10)
    def _():
        off = pl.multiple_of(s * 1000, 8)
        pltpu.sync_copy(zbuf.at[pl.ds(0, 1000)],
                        deg_sh.at[pl.ds(off, 1000)])

    plsc.subcore_barrier()

    def body(ci, carry):
        off = pl.multiple_of(ci * CH, 8)
        pltpu.sync_copy(w_v.at[pl.ds(off, CH)],
                        deg_sh.at[dst_v.at[ci]], add=True)
        return carry

    lax.fori_loop(0, NCHUNK, body, 0)
    plsc.subcore_barrier()

    @pl.when(s < 10)
    def _():
        off = pl.multiple_of(s * 1000, 8)
        oof = pl.multiple_of(c * N + s * 1000, 8)
        pltpu.sync_copy(deg_sh.at[pl.ds(off, 1000)],
                        zbuf.at[pl.ds(0, 1000)])
        pltpu.sync_copy(zbuf.at[pl.ds(0, 1000)],
                        out_hbm.at[pl.ds(oof, 1000)])


def _sc_deg(dst3, w2):
    k = functools.partial(
        pl.kernel,
        mesh=_mesh,
        out_type=jax.ShapeDtypeStruct((NC * N,), jnp.float32),
        scratch_types=[
            pltpu.VMEM((NCHUNK, CH), jnp.int32),
            pltpu.VMEM((EPT,), jnp.float32),
            pltpu.VMEM((1024,), jnp.float32),
            pltpu.VMEM_SHARED((N,), jnp.float32),
        ],
    )(_sc_deg_body)
    return k(dst3, w2)


def _sc_edge_body(src_hbm, dst_hbm, w_hbm, hs_hbm, out_hbm,
                  dst_v, src_v, w_v, rin0, rin1, rout0, rout1,
                  acc_sh, g0, g1, t0, t1, s0, s1):
    c = lax.axis_index("c")
    s = lax.axis_index("s")
    wid = s * NC + c
    rin = (rin0, rin1)
    rout = (rout0, rout1)
    gsem = (g0, g1)
    tsem = (t0, t1)
    ssem = (s0, s1)

    zeros = jnp.zeros((16,), jnp.float32)
    for i in range(CH):
        for j in range(D // 16):
            rout0[i, pl.ds(j * 16, 16)] = zeros

    for k in range(8):
        zi = s + k * NS

        @pl.when(zi < NCHUNK)
        def _():
            off = pl.multiple_of(zi * CH, 8)
            pltpu.sync_copy(rout0, acc_sh.at[pl.ds(off, CH)])

    def fire_stage(ci, b2, b4):
        pltpu.async_copy(src_hbm.at[wid, ci], src_v.at[b2], tsem[b2])
        pltpu.async_copy(dst_hbm.at[wid, ci], dst_v.at[b4], tsem[b2])
        pltpu.async_copy(w_hbm.at[wid, ci], w_v.at[b2], tsem[b2])

    def wait_stage(b2):
        pltpu.make_async_copy(src_hbm.at[0, 0], src_v.at[b2], tsem[b2]).wait()
        pltpu.make_async_copy(src_hbm.at[0, 0], src_v.at[b2], tsem[b2]).wait()
        pltpu.make_async_copy(w_hbm.at[0, 0], w_v.at[b2], tsem[b2]).wait()

    def fire_gather(b2):
        pltpu.async_copy(hs_hbm.at[src_v.at[b2]], rin[b2], gsem[b2])

    def wait_gather(b2):
        pltpu.make_async_copy(hs_hbm.at[src_v.at[b2]], rin[b2],
                              gsem[b2]).wait()

    def wait_scatter(b2):
        pltpu.make_async_copy(rout[b2], acc_sh.at[dst_v.at[0]],
                              ssem[b2]).wait()

    def scale(b2):
        rv = rin[b2]
        ro = rout[b2]

        def group(g, carry):
            w_vec = w_v[b2, pl.ds(g * 16, 16)]
            for r in range(16):
                ws = _splat(w_vec, r)
                for j in range(D // 16):
                    ro[g * 16 + r, pl.ds(j * 16, 16)] = (
                        rv[g * 16 + r, pl.ds(j * 16, 16)] * ws)
            return carry

        lax.fori_loop(0, CH // 16, group, 0)

    def phase(ci, b2, b4):
        bp2 = (b2 + 1) % 2

        @pl.when(ci + 1 < NCHUNK)
        def _():
            wait_stage(bp2)
            fire_gather(bp2)

        wait_gather(b2)

        @pl.when(ci >= 2)
        def _():
            wait_scatter(b2)

        scale(b2)
        pltpu.async_copy(rout[b2], acc_sh.at[dst_v.at[b4]],
                         ssem[b2], add=True)

        @pl.when(ci + 2 < NCHUNK)
        def _():
            fire_stage(ci + 2, b2, (b4 + 2) % 4)

    fire_stage(0, 0, 0)
    fire_stage(1, 1, 1)
    plsc.subcore_barrier()
    wait_stage(0)
    fire_gather(0)

    def quad(k, carry):
        ci = k * 4
        phase(ci, 0, 0)
        phase(ci + 1, 1, 1)
        phase(ci + 2, 0, 2)
        phase(ci + 3, 1, 3)
        return carry

    lax.fori_loop(0, NCHUNK // 4, quad, 0)
    phase(NCHUNK - 1, 0, 0)
    wait_scatter(1)
    wait_scatter(0)
    plsc.subcore_barrier()

    for k in range(8):
        ci = s + k * NS

        @pl.when(ci < NCHUNK)
        def _():
            off = pl.multiple_of(ci * CH, 8)
            oof = pl.multiple_of(c * N + ci * CH, 8)
            pltpu.sync_copy(acc_sh.at[pl.ds(off, CH)], rout0)
            pltpu.sync_copy(rout0, out_hbm.at[pl.ds(oof, CH)])


def _sc_edges(src2, dst3, w2, hs):
    k = functools.partial(
        pl.kernel,
        mesh=_mesh,
        out_type=jax.ShapeDtypeStruct((NC * N, D), jnp.float32),
        scratch_types=[
            pltpu.VMEM((4, CH), jnp.int32),
            pltpu.VMEM((2, CH), jnp.int32),
            pltpu.VMEM((2, CH), jnp.float32),
            pltpu.VMEM((CH, D), jnp.float32),
            pltpu.VMEM((CH, D), jnp.float32),
            pltpu.VMEM((CH, D), jnp.float32),
            pltpu.VMEM((CH, D), jnp.float32),
            pltpu.VMEM_SHARED((N, D), jnp.float32),
        ] + [pltpu.SemaphoreType.DMA] * 6,
    )(_sc_edge_body)
    return k(src2, dst3, w2, hs)


_BLK = 1000


def _tc_mm_body(x_ref, w_ref, degp_ref, h_ref, hs_ref):
    xb = x_ref[...]
    h = jnp.dot(xb, w_ref[...], preferred_element_type=jnp.float32)
    deg = 1.0 + jnp.sum(degp_ref[...], axis=1, keepdims=True)
    dinv = lax.rsqrt(deg)
    h_ref[...] = h
    hs_ref[...] = h * dinv


def _tc_mm(x, W, degp_t):
    grid = (N // _BLK,)
    return pl.pallas_call(
        _tc_mm_body,
        grid=grid,
        in_specs=[
            pl.BlockSpec((_BLK, D), lambda i: (i, 0)),
            pl.BlockSpec((D, D), lambda i: (0, 0)),
            pl.BlockSpec((_BLK, NC), lambda i: (i, 0)),
        ],
        out_specs=[
            pl.BlockSpec((_BLK, D), lambda i: (i, 0)),
            pl.BlockSpec((_BLK, D), lambda i: (i, 0)),
        ],
        out_shape=[
            jax.ShapeDtypeStruct((N, D), jnp.float32),
            jax.ShapeDtypeStruct((N, D), jnp.float32),
        ],
    )(x, W, degp_t)


def _tc_final_body(acc_ref, h_ref, degp_ref, b_ref, g_ref, be_ref, o_ref):
    acc = acc_ref[0] + acc_ref[1]
    deg = 1.0 + jnp.sum(degp_ref[...], axis=1, keepdims=True)
    dinv = lax.rsqrt(deg)
    pre = acc * dinv + h_ref[...] * (dinv * dinv) + b_ref[...]
    mean = jnp.mean(pre, axis=0, keepdims=True)
    var = jnp.mean((pre - mean) * (pre - mean), axis=0, keepdims=True)
    o = (pre - mean) * lax.rsqrt(var + 1e-5) * g_ref[...] + be_ref[...]
    o_ref[...] = jnp.maximum(o, 0.0)


def _tc_final(acc, h, degp_t, b, gamma, beta):
    return pl.pallas_call(
        _tc_final_body,
        out_shape=jax.ShapeDtypeStruct((N, D), jnp.float32),
    )(acc, h, degp_t, b, gamma, beta)


def kernel(x, edge_index, edge_weight, W, b, gamma, beta):
    src = edge_index[0]
    dst = edge_index[1]
    src3 = src.reshape(NW, NCHUNK, CH)
    dst3 = dst.reshape(NW, NCHUNK, CH)
    w3 = edge_weight.reshape(NW, NCHUNK, CH)
    w2 = edge_weight.reshape(NW, 1, EPT)

    degp = _sc_deg(dst3, w2).reshape(NC, N)
    degp_t = degp.T
    h, hs = _tc_mm(x, W, degp_t)
    acc = _sc_edges(src3, dst3, w3, hs).reshape(NC, N, D)
    out = _tc_final(acc, h, degp_t,
                    b.reshape(1, D), gamma.reshape(1, D), beta.reshape(1, D))
    return out


# speedup vs baseline: 1.4447x; 1.4447x over previous
import functools

import jax
import jax.numpy as jnp
from jax import lax
from jax.experimental import pallas as pl
from jax.experimental.pallas import tpu as pltpu
from jax.experimental.pallas import tpu_sc as plsc

N = 10000
E = 320000
D = 128

NC = 2
NS = 16
NW = NC * NS
EPT = E // NW
CH = 80
NCHUNK = EPT // CH
RPT = N // NS

_mesh = plsc.VectorSubcoreMesh(core_axis_name="c", subcore_axis_name="s")

_GD = lax.GatherDimensionNumbers(
    offset_dims=(), collapsed_slice_dims=(0,), start_index_map=(0,))


def _splat(v, r):
    idx = jnp.full((16, 1), r, jnp.int32)
    return lax.gather(v, idx, _GD, (1,),
                      mode=lax.GatherScatterMode.PROMISE_IN_BOUNDS)


def _sc_deg_body(dst_hbm, w_hbm, out_hbm, dst_v, w_v, zbuf, deg_sh):
    c = lax.axis_index("c")
    s = lax.axis_index("s")
    wid = s * NC + c
    pltpu.sync_copy(dst_hbm.at[wid], dst_v)
    pltpu.sync_copy(w_hbm.at[wid, 0], w_v)

    zeros = jnp.zeros((16,), jnp.float32)

    def zero_body(i, carry):
        zbuf[pl.ds(i * 16, 16)] = zeros
        return carry

    lax.fori_loop(0, 1024 // 16, zero_body, 0)

    @pl.when(s < 10)
    def _():
        off = pl.multiple_of(s * 1000, 8)
        pltpu.sync_copy(zbuf.at[pl.ds(0, 1000)],
                        deg_sh.at[pl.ds(off, 1000)])

    plsc.subcore_barrier()

    def body(ci, carry):
        off = pl.multiple_of(ci * CH, 8)
        pltpu.sync_copy(w_v.at[pl.ds(off, CH)],
                        deg_sh.at[dst_v.at[ci]], add=True)
        return carry

    lax.fori_loop(0, NCHUNK, body, 0)
    plsc.subcore_barrier()

    @pl.when(s < 10)
    def _():
        off = pl.multiple_of(s * 1000, 8)
        oof = pl.multiple_of(c * N + s * 1000, 8)
        pltpu.sync_copy(deg_sh.at[pl.ds(off, 1000)],
                        zbuf.at[pl.ds(0, 1000)])
        pltpu.sync_copy(zbuf.at[pl.ds(0, 1000)],
                        out_hbm.at[pl.ds(oof, 1000)])


def _sc_deg(dst3, w2):
    k = functools.partial(
        pl.kernel,
        mesh=_mesh,
        out_type=jax.ShapeDtypeStruct((NC * N,), jnp.float32),
        scratch_types=[
            pltpu.VMEM((NCHUNK, CH), jnp.int32),
            pltpu.VMEM((EPT,), jnp.float32),
            pltpu.VMEM((1024,), jnp.float32),
            pltpu.VMEM_SHARED((N,), jnp.float32),
        ],
    )(_sc_deg_body)
    return k(dst3, w2)


def _sc_edge_body(src_hbm, dst_hbm, w_hbm, hs_hbm, out_hbm,
                  dst_v, src_v, w_v, rin0, rin1, rout0, rout1,
                  acc_sh, g0, g1, t0, t1, s0, s1):
    c = lax.axis_index("c")
    s = lax.axis_index("s")
    wid = s * NC + c
    rin = (rin0, rin1)
    rout = (rout0, rout1)
    gsem = (g0, g1)
    tsem = (t0, t1)
    ssem = (s0, s1)

    zeros = jnp.zeros((16,), jnp.float32)
    for i in range(CH):
        for j in range(D // 16):
            rout0[i, pl.ds(j * 16, 16)] = zeros

    for k in range(8):
        zi = s + k * NS

        @pl.when(zi < NCHUNK)
        def _():
            off = pl.multiple_of(zi * CH, 8)
            pltpu.sync_copy(rout0, acc_sh.at[pl.ds(off, CH)])

    def fire_stage(ci, b2, b4):
        pltpu.async_copy(src_hbm.at[wid, ci], src_v.at[b2], tsem[b2])
        pltpu.async_copy(dst_hbm.at[wid, ci], dst_v.at[b4], tsem[b2])
        pltpu.async_copy(w_hbm.at[wid, ci], w_v.at[b2], tsem[b2])

    def wait_stage(b2):
        pltpu.make_async_copy(src_hbm.at[0, 0], src_v.at[b2], tsem[b2]).wait()
        pltpu.make_async_copy(src_hbm.at[0, 0], src_v.at[b2], tsem[b2]).wait()
        pltpu.make_async_copy(w_hbm.at[0, 0], w_v.at[b2], tsem[b2]).wait()

    def fire_gather(b2):
        pltpu.async_copy(hs_hbm.at[src_v.at[b2]], rin[b2], gsem[b2])

    def wait_gather(b2):
        pltpu.make_async_copy(hs_hbm.at[src_v.at[b2]], rin[b2],
                              gsem[b2]).wait()

    def wait_scatter(b2):
        pltpu.make_async_copy(rout[b2], acc_sh.at[dst_v.at[0]],
                              ssem[b2]).wait()

    def scale(b2):
        rv = rin[b2]
        ro = rout[b2]
        for g in range(CH // 16):
            w_vec = w_v[b2, pl.ds(g * 16, 16)]
            for r in range(16):
                e = g * 16 + r
                ws = _splat(w_vec, r)
                for j in range(D // 16):
                    ro[e, pl.ds(j * 16, 16)] = (
                        rv[e, pl.ds(j * 16, 16)] * ws)

    def phase(ci, b2, b4):
        bp2 = (b2 + 1) % 2

        @pl.when(ci + 1 < NCHUNK)
        def _():
            wait_stage(bp2)
            fire_gather(bp2)

        wait_gather(b2)

        @pl.when(ci >= 2)
        def _():
            wait_scatter(b2)

        scale(b2)
        pltpu.async_copy(rout[b2], acc_sh.at[dst_v.at[b4]],
                         ssem[b2], add=True)

        @pl.when(ci + 2 < NCHUNK)
        def _():
            fire_stage(ci + 2, b2, (b4 + 2) % 4)

    fire_stage(0, 0, 0)
    fire_stage(1, 1, 1)
    plsc.subcore_barrier()
    wait_stage(0)
    fire_gather(0)

    def pair(k, carry):
        ci = k * 2
        b4a = (k % 2) * 2
        phase(ci, 0, b4a)
        phase(ci + 1, 1, b4a + 1)
        return carry

    lax.fori_loop(0, NCHUNK // 2, pair, 0)
    phase(NCHUNK - 1, 0, 0)
    wait_scatter(1)
    wait_scatter(0)
    plsc.subcore_barrier()

    for k in range(8):
        ci = s + k * NS

        @pl.when(ci < NCHUNK)
        def _():
            off = pl.multiple_of(ci * CH, 8)
            oof = pl.multiple_of(c * N + ci * CH, 8)
            pltpu.sync_copy(acc_sh.at[pl.ds(off, CH)], rout0)
            pltpu.sync_copy(rout0, out_hbm.at[pl.ds(oof, CH)])


def _sc_edges(src2, dst3, w2, hs):
    k = functools.partial(
        pl.kernel,
        mesh=_mesh,
        out_type=jax.ShapeDtypeStruct((NC * N, D), jnp.float32),
        scratch_types=[
            pltpu.VMEM((4, CH), jnp.int32),
            pltpu.VMEM((2, CH), jnp.int32),
            pltpu.VMEM((2, CH), jnp.float32),
            pltpu.VMEM((CH, D), jnp.float32),
            pltpu.VMEM((CH, D), jnp.float32),
            pltpu.VMEM((CH, D), jnp.float32),
            pltpu.VMEM((CH, D), jnp.float32),
            pltpu.VMEM_SHARED((N, D), jnp.float32),
        ] + [pltpu.SemaphoreType.DMA] * 6,
    )(_sc_edge_body)
    return k(src2, dst3, w2, hs)


_BLK = 1000


def _tc_mm_body(x_ref, w_ref, degp_ref, h_ref, hs_ref):
    xb = x_ref[...]
    h = jnp.dot(xb, w_ref[...], preferred_element_type=jnp.float32)
    deg = 1.0 + jnp.sum(degp_ref[...], axis=1, keepdims=True)
    dinv = lax.rsqrt(deg)
    h_ref[...] = h
    hs_ref[...] = h * dinv


def _tc_mm(x, W, degp_t):
    grid = (N // _BLK,)
    return pl.pallas_call(
        _tc_mm_body,
        grid=grid,
        in_specs=[
            pl.BlockSpec((_BLK, D), lambda i: (i, 0)),
            pl.BlockSpec((D, D), lambda i: (0, 0)),
            pl.BlockSpec((_BLK, NC), lambda i: (i, 0)),
        ],
        out_specs=[
            pl.BlockSpec((_BLK, D), lambda i: (i, 0)),
            pl.BlockSpec((_BLK, D), lambda i: (i, 0)),
        ],
        out_shape=[
            jax.ShapeDtypeStruct((N, D), jnp.float32),
            jax.ShapeDtypeStruct((N, D), jnp.float32),
        ],
    )(x, W, degp_t)


def _tc_final_body(acc_ref, h_ref, degp_ref, b_ref, g_ref, be_ref, o_ref):
    acc = acc_ref[0] + acc_ref[1]
    deg = 1.0 + jnp.sum(degp_ref[...], axis=1, keepdims=True)
    dinv = lax.rsqrt(deg)
    pre = acc * dinv + h_ref[...] * (dinv * dinv) + b_ref[...]
    mean = jnp.mean(pre, axis=0, keepdims=True)
    var = jnp.mean((pre - mean) * (pre - mean), axis=0, keepdims=True)
    o = (pre - mean) * lax.rsqrt(var + 1e-5) * g_ref[...] + be_ref[...]
    o_ref[...] = jnp.maximum(o, 0.0)


def _tc_final(acc, h, degp_t, b, gamma, beta):
    return pl.pallas_call(
        _tc_final_body,
        out_shape=jax.ShapeDtypeStruct((N, D), jnp.float32),
    )(acc, h, degp_t, b, gamma, beta)


def kernel(x, edge_index, edge_weight, W, b, gamma, beta):
    src = edge_index[0]
    dst = edge_index[1]
    src3 = src.reshape(NW, NCHUNK, CH)
    dst3 = dst.reshape(NW, NCHUNK, CH)
    w3 = edge_weight.reshape(NW, NCHUNK, CH)
    w2 = edge_weight.reshape(NW, 1, EPT)

    degp = _sc_deg(dst3, w2).reshape(NC, N)
    degp_t = degp.T
    h, hs = _tc_mm(x, W, degp_t)
    acc = _sc_edges(src3, dst3, w3, hs).reshape(NC, N, D)
    out = _tc_final(acc, h, degp_t,
                    b.reshape(1, D), gamma.reshape(1, D), beta.reshape(1, D))
    return out


# speedup vs baseline: 1.4830x; 1.0265x over previous
import functools

import jax
import jax.numpy as jnp
from jax import lax
from jax.experimental import pallas as pl
from jax.experimental.pallas import tpu as pltpu
from jax.experimental.pallas import tpu_sc as plsc

N = 10000
E = 320000
D = 128

NC = 2
NS = 16
NW = NC * NS
EPT = E // NW
CH = 80
NCHUNK = EPT // CH
RPT = N // NS

_mesh = plsc.VectorSubcoreMesh(core_axis_name="c", subcore_axis_name="s")

_GD = lax.GatherDimensionNumbers(
    offset_dims=(), collapsed_slice_dims=(0,), start_index_map=(0,))


def _splat(v, r):
    idx = jnp.full((16, 1), r, jnp.int32)
    return lax.gather(v, idx, _GD, (1,),
                      mode=lax.GatherScatterMode.PROMISE_IN_BOUNDS)


def _sc_deg_body(dst_hbm, w_hbm, out_hbm, dst_v, w_v, zbuf, deg_sh):
    c = lax.axis_index("c")
    s = lax.axis_index("s")
    wid = s * NC + c
    pltpu.sync_copy(dst_hbm.at[wid], dst_v)
    pltpu.sync_copy(w_hbm.at[wid, 0], w_v)

    zeros = jnp.zeros((16,), jnp.float32)

    def zero_body(i, carry):
        zbuf[pl.ds(i * 16, 16)] = zeros
        return carry

    lax.fori_loop(0, 1024 // 16, zero_body, 0)

    @pl.when(s < 10)
    def _():
        off = pl.multiple_of(s * 1000, 8)
        pltpu.sync_copy(zbuf.at[pl.ds(0, 1000)],
                        deg_sh.at[pl.ds(off, 1000)])

    plsc.subcore_barrier()

    def body(ci, carry):
        off = pl.multiple_of(ci * CH, 8)
        pltpu.sync_copy(w_v.at[pl.ds(off, CH)],
                        deg_sh.at[dst_v.at[ci]], add=True)
        return carry

    lax.fori_loop(0, NCHUNK, body, 0)
    plsc.subcore_barrier()

    @pl.when(s < 10)
    def _():
        off = pl.multiple_of(s * 1000, 8)
        oof = pl.multiple_of(c * N + s * 1000, 8)
        pltpu.sync_copy(deg_sh.at[pl.ds(off, 1000)],
                        zbuf.at[pl.ds(0, 1000)])
        pltpu.sync_copy(zbuf.at[pl.ds(0, 1000)],
                        out_hbm.at[pl.ds(oof, 1000)])


def _sc_deg(dst3, w2):
    k = functools.partial(
        pl.kernel,
        mesh=_mesh,
        out_type=jax.ShapeDtypeStruct((NC * N,), jnp.float32),
        scratch_types=[
            pltpu.VMEM((NCHUNK, CH), jnp.int32),
            pltpu.VMEM((EPT,), jnp.float32),
            pltpu.VMEM((1024,), jnp.float32),
            pltpu.VMEM_SHARED((N,), jnp.float32),
        ],
    )(_sc_deg_body)
    return k(dst3, w2)


def _sc_edge_body(src_hbm, dst_hbm, w_hbm, hs_hbm, out_hbm,
                  dst_v, src_v, w_v, rin0, rin1, rout0, rout1,
                  acc_sh, g0, g1, h0, h1, t0, t1, s0, s1):
    c = lax.axis_index("c")
    s = lax.axis_index("s")
    wid = s * NC + c
    rin = (rin0, rin1)
    rout = (rout0, rout1)
    gsem = (g0, g1)
    hsem = (h0, h1)
    tsem = (t0, t1)
    ssem = (s0, s1)

    zeros = jnp.zeros((16,), jnp.float32)
    for i in range(CH):
        for j in range(D // 16):
            rout0[i, pl.ds(j * 16, 16)] = zeros

    for k in range(8):
        zi = s + k * NS

        @pl.when(zi < NCHUNK)
        def _():
            off = pl.multiple_of(zi * CH, 8)
            pltpu.sync_copy(rout0, acc_sh.at[pl.ds(off, CH)])

    def fire_stage(ci, b2, b4):
        pltpu.async_copy(src_hbm.at[wid, ci], src_v.at[b2], tsem[b2])
        pltpu.async_copy(dst_hbm.at[wid, ci], dst_v.at[b4], tsem[b2])
        pltpu.async_copy(w_hbm.at[wid, ci], w_v.at[b2], tsem[b2])

    def wait_stage(b2):
        pltpu.make_async_copy(src_hbm.at[0, 0], src_v.at[b2], tsem[b2]).wait()
        pltpu.make_async_copy(src_hbm.at[0, 0], src_v.at[b2], tsem[b2]).wait()
        pltpu.make_async_copy(w_hbm.at[0, 0], w_v.at[b2], tsem[b2]).wait()

    H1 = 48
    H2 = CH - H1

    def fire_gather(b2):
        pltpu.async_copy(hs_hbm.at[src_v.at[b2, pl.ds(0, H1)]],
                         rin[b2].at[pl.ds(0, H1)], gsem[b2])
        pltpu.async_copy(hs_hbm.at[src_v.at[b2, pl.ds(H1, H2)]],
                         rin[b2].at[pl.ds(H1, H2)], hsem[b2])

    def wait_gather_half(b2, half):
        if half == 0:
            pltpu.make_async_copy(hs_hbm.at[src_v.at[b2, pl.ds(0, H1)]],
                                  rin[b2].at[pl.ds(0, H1)],
                                  gsem[b2]).wait()
        else:
            pltpu.make_async_copy(hs_hbm.at[src_v.at[b2, pl.ds(H1, H2)]],
                                  rin[b2].at[pl.ds(H1, H2)],
                                  hsem[b2]).wait()

    def wait_scatter(b2):
        pltpu.make_async_copy(rout[b2], acc_sh.at[dst_v.at[0]],
                              ssem[b2]).wait()

    def scale(b2, g_lo, g_hi):
        rv = rin[b2]
        ro = rout[b2]
        for g in range(g_lo, g_hi):
            w_vec = w_v[b2, pl.ds(g * 16, 16)]
            for r in range(16):
                e = g * 16 + r
                ws = _splat(w_vec, r)
                for j in range(D // 16):
                    ro[e, pl.ds(j * 16, 16)] = (
                        rv[e, pl.ds(j * 16, 16)] * ws)

    def phase(ci, b2, b4):
        bp2 = (b2 + 1) % 2

        @pl.when(ci + 1 < NCHUNK)
        def _():
            wait_stage(bp2)
            fire_gather(bp2)

        wait_gather_half(b2, 0)

        @pl.when(ci >= 2)
        def _():
            wait_scatter(b2)

        scale(b2, 0, H1 // 16)
        wait_gather_half(b2, 1)
        scale(b2, H1 // 16, CH // 16)
        pltpu.async_copy(rout[b2], acc_sh.at[dst_v.at[b4]],
                         ssem[b2], add=True)

        @pl.when(ci + 2 < NCHUNK)
        def _():
            fire_stage(ci + 2, b2, (b4 + 2) % 4)

    fire_stage(0, 0, 0)
    fire_stage(1, 1, 1)
    plsc.subcore_barrier()
    wait_stage(0)
    fire_gather(0)

    def pair(k, carry):
        ci = k * 2
        b4a = (k % 2) * 2
        phase(ci, 0, b4a)
        phase(ci + 1, 1, b4a + 1)
        return carry

    lax.fori_loop(0, NCHUNK // 2, pair, 0)
    phase(NCHUNK - 1, 0, 0)
    wait_scatter(1)
    wait_scatter(0)
    plsc.subcore_barrier()

    for k in range(8):
        ci = s + k * NS

        @pl.when(ci < NCHUNK)
        def _():
            off = pl.multiple_of(ci * CH, 8)
            oof = pl.multiple_of(c * N + ci * CH, 8)
            pltpu.sync_copy(acc_sh.at[pl.ds(off, CH)], rout0)
            pltpu.sync_copy(rout0, out_hbm.at[pl.ds(oof, CH)])


def _sc_edges(src2, dst3, w2, hs):
    k = functools.partial(
        pl.kernel,
        mesh=_mesh,
        out_type=jax.ShapeDtypeStruct((NC * N, D), jnp.float32),
        scratch_types=[
            pltpu.VMEM((4, CH), jnp.int32),
            pltpu.VMEM((2, CH), jnp.int32),
            pltpu.VMEM((2, CH), jnp.float32),
            pltpu.VMEM((CH, D), jnp.float32),
            pltpu.VMEM((CH, D), jnp.float32),
            pltpu.VMEM((CH, D), jnp.float32),
            pltpu.VMEM((CH, D), jnp.float32),
            pltpu.VMEM_SHARED((N, D), jnp.float32),
        ] + [pltpu.SemaphoreType.DMA] * 8,
    )(_sc_edge_body)
    return k(src2, dst3, w2, hs)


_BLK = 1000


def _tc_mm_body(x_ref, w_ref, degp_ref, h_ref, hs_ref):
    xb = x_ref[...]
    h = jnp.dot(xb, w_ref[...], preferred_element_type=jnp.float32)
    deg = 1.0 + jnp.sum(degp_ref[...], axis=1, keepdims=True)
    dinv = lax.rsqrt(deg)
    h_ref[...] = h
    hs_ref[...] = h * dinv


def _tc_mm(x, W, degp_t):
    grid = (N // _BLK,)
    return pl.pallas_call(
        _tc_mm_body,
        grid=grid,
        in_specs=[
            pl.BlockSpec((_BLK, D), lambda i: (i, 0)),
            pl.BlockSpec((D, D), lambda i: (0, 0)),
            pl.BlockSpec((_BLK, NC), lambda i: (i, 0)),
        ],
        out_specs=[
            pl.BlockSpec((_BLK, D), lambda i: (i, 0)),
            pl.BlockSpec((_BLK, D), lambda i: (i, 0)),
        ],
        out_shape=[
            jax.ShapeDtypeStruct((N, D), jnp.float32),
            jax.ShapeDtypeStruct((N, D), jnp.float32),
        ],
    )(x, W, degp_t)


def _tc_final_body(acc_ref, h_ref, degp_ref, b_ref, g_ref, be_ref, o_ref):
    acc = acc_ref[0] + acc_ref[1]
    deg = 1.0 + jnp.sum(degp_ref[...], axis=1, keepdims=True)
    dinv = lax.rsqrt(deg)
    pre = acc * dinv + h_ref[...] * (dinv * dinv) + b_ref[...]
    mean = jnp.mean(pre, axis=0, keepdims=True)
    var = jnp.mean((pre - mean) * (pre - mean), axis=0, keepdims=True)
    o = (pre - mean) * lax.rsqrt(var + 1e-5) * g_ref[...] + be_ref[...]
    o_ref[...] = jnp.maximum(o, 0.0)


def _tc_final(acc, h, degp_t, b, gamma, beta):
    return pl.pallas_call(
        _tc_final_body,
        out_shape=jax.ShapeDtypeStruct((N, D), jnp.float32),
    )(acc, h, degp_t, b, gamma, beta)


def kernel(x, edge_index, edge_weight, W, b, gamma, beta):
    src = edge_index[0]
    dst = edge_index[1]
    src3 = src.reshape(NW, NCHUNK, CH)
    dst3 = dst.reshape(NW, NCHUNK, CH)
    w3 = edge_weight.reshape(NW, NCHUNK, CH)
    w2 = edge_weight.reshape(NW, 1, EPT)

    degp = _sc_deg(dst3, w2).reshape(NC, N)
    degp_t = degp.T
    h, hs = _tc_mm(x, W, degp_t)
    acc = _sc_edges(src3, dst3, w3, hs).reshape(NC, N, D)
    out = _tc_final(acc, h, degp_t,
                    b.reshape(1, D), gamma.reshape(1, D), beta.reshape(1, D))
    return out


# speedup vs baseline: 1.5139x; 1.0209x over previous
import functools

import jax
import jax.numpy as jnp
from jax import lax
from jax.experimental import pallas as pl
from jax.experimental.pallas import tpu as pltpu
from jax.experimental.pallas import tpu_sc as plsc

N = 10000
E = 320000
D = 128

NC = 2
NS = 16
NW = NC * NS
EPT = E // NW
CH = 80
NCHUNK = EPT // CH
RPT = N // NS

_mesh = plsc.VectorSubcoreMesh(core_axis_name="c", subcore_axis_name="s")

_GD = lax.GatherDimensionNumbers(
    offset_dims=(), collapsed_slice_dims=(0,), start_index_map=(0,))


def _splat(v, r):
    idx = jnp.full((16, 1), r, jnp.int32)
    return lax.gather(v, idx, _GD, (1,),
                      mode=lax.GatherScatterMode.PROMISE_IN_BOUNDS)


def _sc_deg_body(dst_hbm, w_hbm, out_hbm, dst_v, w_v, zbuf, deg_sh, sem):
    c = lax.axis_index("c")
    s = lax.axis_index("s")
    wid = s * NC + c
    pltpu.sync_copy(dst_hbm.at[wid], dst_v)
    pltpu.sync_copy(w_hbm.at[wid, 0], w_v)

    zeros = jnp.zeros((16,), jnp.float32)

    def zero_body(i, carry):
        zbuf[pl.ds(i * 16, 16)] = zeros
        return carry

    lax.fori_loop(0, 1024 // 16, zero_body, 0)

    @pl.when(s < 10)
    def _():
        off = pl.multiple_of(s * 1000, 8)
        pltpu.sync_copy(zbuf.at[pl.ds(0, 1000)],
                        deg_sh.at[pl.ds(off, 1000)])

    plsc.subcore_barrier()

    def fire(ci):
        off = pl.multiple_of(ci * CH, 8)
        pltpu.async_copy(w_v.at[pl.ds(off, CH)],
                         deg_sh.at[dst_v.at[ci]], sem, add=True)

    def wait_one():
        pltpu.make_async_copy(w_v.at[pl.ds(0, CH)],
                              deg_sh.at[dst_v.at[0]], sem).wait()

    fire(0)
    fire(1)

    def body(ci, carry):
        wait_one()
        fire(ci)
        return carry

    lax.fori_loop(2, NCHUNK, body, 0)
    wait_one()
    wait_one()
    plsc.subcore_barrier()

    @pl.when(s < 10)
    def _():
        off = pl.multiple_of(s * 1000, 8)
        oof = pl.multiple_of(c * N + s * 1000, 8)
        pltpu.sync_copy(deg_sh.at[pl.ds(off, 1000)],
                        zbuf.at[pl.ds(0, 1000)])
        pltpu.sync_copy(zbuf.at[pl.ds(0, 1000)],
                        out_hbm.at[pl.ds(oof, 1000)])


def _sc_deg(dst3, w2):
    k = functools.partial(
        pl.kernel,
        mesh=_mesh,
        out_type=jax.ShapeDtypeStruct((NC * N,), jnp.float32),
        scratch_types=[
            pltpu.VMEM((NCHUNK, CH), jnp.int32),
            pltpu.VMEM((EPT,), jnp.float32),
            pltpu.VMEM((1024,), jnp.float32),
            pltpu.VMEM_SHARED((N,), jnp.float32),
            pltpu.SemaphoreType.DMA,
        ],
    )(_sc_deg_body)
    return k(dst3, w2)


def _sc_edge_body(src_hbm, dst_hbm, w_hbm, hs_hbm, out_hbm,
                  dst_v, src_v, w_v, rin0, rin1, rout0, rout1,
                  acc_sh, g0, g1, h0, h1, t0, t1, s0, s1):
    c = lax.axis_index("c")
    s = lax.axis_index("s")
    wid = s * NC + c
    rin = (rin0, rin1)
    rout = (rout0, rout1)
    gsem = (g0, g1)
    hsem = (h0, h1)
    tsem = (t0, t1)
    ssem = (s0, s1)

    zeros = jnp.zeros((16,), jnp.float32)
    for i in range(CH):
        for j in range(D // 16):
            rout0[i, pl.ds(j * 16, 16)] = zeros

    for k in range(8):
        zi = s + k * NS

        @pl.when(zi < NCHUNK)
        def _():
            off = pl.multiple_of(zi * CH, 8)
            pltpu.sync_copy(rout0, acc_sh.at[pl.ds(off, CH)])

    def fire_stage(ci, b2, b4):
        pltpu.async_copy(src_hbm.at[wid, ci], src_v.at[b2], tsem[b2])
        pltpu.async_copy(dst_hbm.at[wid, ci], dst_v.at[b4], tsem[b2])
        pltpu.async_copy(w_hbm.at[wid, ci], w_v.at[b2], tsem[b2])

    def wait_stage(b2):
        pltpu.make_async_copy(src_hbm.at[0, 0], src_v.at[b2], tsem[b2]).wait()
        pltpu.make_async_copy(src_hbm.at[0, 0], src_v.at[b2], tsem[b2]).wait()
        pltpu.make_async_copy(w_hbm.at[0, 0], w_v.at[b2], tsem[b2]).wait()

    H1 = 48
    H2 = CH - H1

    def fire_gather(b2):
        pltpu.async_copy(hs_hbm.at[src_v.at[b2, pl.ds(0, H1)]],
                         rin[b2].at[pl.ds(0, H1)], gsem[b2])
        pltpu.async_copy(hs_hbm.at[src_v.at[b2, pl.ds(H1, H2)]],
                         rin[b2].at[pl.ds(H1, H2)], hsem[b2])

    def wait_gather_half(b2, half):
        if half == 0:
            pltpu.make_async_copy(hs_hbm.at[src_v.at[b2, pl.ds(0, H1)]],
                                  rin[b2].at[pl.ds(0, H1)],
                                  gsem[b2]).wait()
        else:
            pltpu.make_async_copy(hs_hbm.at[src_v.at[b2, pl.ds(H1, H2)]],
                                  rin[b2].at[pl.ds(H1, H2)],
                                  hsem[b2]).wait()

    def wait_scatter(b2):
        pltpu.make_async_copy(rout[b2], acc_sh.at[dst_v.at[0]],
                              ssem[b2]).wait()

    def scale(b2, g_lo, g_hi):
        rv = rin[b2]
        ro = rout[b2]
        for g in range(g_lo, g_hi):
            w_vec = w_v[b2, pl.ds(g * 16, 16)]
            for r in range(16):
                e = g * 16 + r
                ws = _splat(w_vec, r)
                for j in range(D // 16):
                    ro[e, pl.ds(j * 16, 16)] = (
                        rv[e, pl.ds(j * 16, 16)] * ws)

    def phase(ci, b2, b4):
        bp2 = (b2 + 1) % 2

        @pl.when(ci + 1 < NCHUNK)
        def _():
            wait_stage(bp2)
            fire_gather(bp2)

        wait_gather_half(b2, 0)

        @pl.when(ci >= 2)
        def _():
            wait_scatter(b2)

        scale(b2, 0, H1 // 16)
        wait_gather_half(b2, 1)
        scale(b2, H1 // 16, CH // 16)
        pltpu.async_copy(rout[b2], acc_sh.at[dst_v.at[b4]],
                         ssem[b2], add=True)

        @pl.when(ci + 2 < NCHUNK)
        def _():
            fire_stage(ci + 2, b2, (b4 + 2) % 4)

    fire_stage(0, 0, 0)
    fire_stage(1, 1, 1)
    plsc.subcore_barrier()
    wait_stage(0)
    fire_gather(0)

    def pair(k, carry):
        ci = k * 2
        b4a = (k % 2) * 2
        phase(ci, 0, b4a)
        phase(ci + 1, 1, b4a + 1)
        return carry

    lax.fori_loop(0, NCHUNK // 2, pair, 0)
    phase(NCHUNK - 1, 0, 0)
    wait_scatter(1)
    wait_scatter(0)
    plsc.subcore_barrier()

    for k in range(8):
        ci = s + k * NS

        @pl.when(ci < NCHUNK)
        def _():
            off = pl.multiple_of(ci * CH, 8)
            oof = pl.multiple_of(c * N + ci * CH, 8)
            pltpu.sync_copy(acc_sh.at[pl.ds(off, CH)], rout0)
            pltpu.sync_copy(rout0, out_hbm.at[pl.ds(oof, CH)])


def _sc_edges(src2, dst3, w2, hs):
    k = functools.partial(
        pl.kernel,
        mesh=_mesh,
        out_type=jax.ShapeDtypeStruct((NC * N, D), jnp.float32),
        scratch_types=[
            pltpu.VMEM((4, CH), jnp.int32),
            pltpu.VMEM((2, CH), jnp.int32),
            pltpu.VMEM((2, CH), jnp.float32),
            pltpu.VMEM((CH, D), jnp.float32),
            pltpu.VMEM((CH, D), jnp.float32),
            pltpu.VMEM((CH, D), jnp.float32),
            pltpu.VMEM((CH, D), jnp.float32),
            pltpu.VMEM_SHARED((N, D), jnp.float32),
        ] + [pltpu.SemaphoreType.DMA] * 8,
    )(_sc_edge_body)
    return k(src2, dst3, w2, hs)


_BLK = 1000


def _tc_mm_body(x_ref, w_ref, degp_ref, h_ref, hs_ref):
    xb = x_ref[...]
    h = jnp.dot(xb, w_ref[...], preferred_element_type=jnp.float32)
    deg = 1.0 + jnp.sum(degp_ref[...], axis=1, keepdims=True)
    dinv = lax.rsqrt(deg)
    h_ref[...] = h
    hs_ref[...] = h * dinv


def _tc_mm(x, W, degp_t):
    grid = (N // _BLK,)
    return pl.pallas_call(
        _tc_mm_body,
        grid=grid,
        in_specs=[
            pl.BlockSpec((_BLK, D), lambda i: (i, 0)),
            pl.BlockSpec((D, D), lambda i: (0, 0)),
            pl.BlockSpec((_BLK, NC), lambda i: (i, 0)),
        ],
        out_specs=[
            pl.BlockSpec((_BLK, D), lambda i: (i, 0)),
            pl.BlockSpec((_BLK, D), lambda i: (i, 0)),
        ],
        out_shape=[
            jax.ShapeDtypeStruct((N, D), jnp.float32),
            jax.ShapeDtypeStruct((N, D), jnp.float32),
        ],
    )(x, W, degp_t)


def _tc_final_body(acc_ref, h_ref, degp_ref, b_ref, g_ref, be_ref, o_ref):
    acc = acc_ref[0] + acc_ref[1]
    deg = 1.0 + jnp.sum(degp_ref[...], axis=1, keepdims=True)
    dinv = lax.rsqrt(deg)
    pre = acc * dinv + h_ref[...] * (dinv * dinv) + b_ref[...]
    mean = jnp.mean(pre, axis=0, keepdims=True)
    var = jnp.mean((pre - mean) * (pre - mean), axis=0, keepdims=True)
    o = (pre - mean) * lax.rsqrt(var + 1e-5) * g_ref[...] + be_ref[...]
    o_ref[...] = jnp.maximum(o, 0.0)


def _tc_final(acc, h, degp_t, b, gamma, beta):
    return pl.pallas_call(
        _tc_final_body,
        out_shape=jax.ShapeDtypeStruct((N, D), jnp.float32),
    )(acc, h, degp_t, b, gamma, beta)


def kernel(x, edge_index, edge_weight, W, b, gamma, beta):
    src = edge_index[0]
    dst = edge_index[1]
    src3 = src.reshape(NW, NCHUNK, CH)
    dst3 = dst.reshape(NW, NCHUNK, CH)
    w3 = edge_weight.reshape(NW, NCHUNK, CH)
    w2 = edge_weight.reshape(NW, 1, EPT)

    degp = _sc_deg(dst3, w2).reshape(NC, N)
    degp_t = degp.T
    h, hs = _tc_mm(x, W, degp_t)
    acc = _sc_edges(src3, dst3, w3, hs).reshape(NC, N, D)
    out = _tc_final(acc, h, degp_t,
                    b.reshape(1, D), gamma.reshape(1, D), beta.reshape(1, D))
    return out


# speedup vs baseline: 1.5299x; 1.0106x over previous
import functools

import jax
import jax.numpy as jnp
from jax import lax
from jax.experimental import pallas as pl
from jax.experimental.pallas import tpu as pltpu
from jax.experimental.pallas import tpu_sc as plsc

N = 10000
E = 320000
D = 128

NC = 2
NS = 16
NW = NC * NS
EPT = E // NW
CH = 80
NCHUNK = EPT // CH
RPT = N // NS

_mesh = plsc.VectorSubcoreMesh(core_axis_name="c", subcore_axis_name="s")

_GD = lax.GatherDimensionNumbers(
    offset_dims=(), collapsed_slice_dims=(0,), start_index_map=(0,))


def _splat(v, r):
    idx = jnp.full((16, 1), r, jnp.int32)
    return lax.gather(v, idx, _GD, (1,),
                      mode=lax.GatherScatterMode.PROMISE_IN_BOUNDS)


def _sc_deg_body(dst_hbm, w_hbm, out_hbm, dst_v, w_v, zbuf, deg_sh, sem):
    c = lax.axis_index("c")
    s = lax.axis_index("s")
    wid = s * NC + c
    pltpu.sync_copy(dst_hbm.at[wid], dst_v)
    pltpu.sync_copy(w_hbm.at[wid, 0], w_v)

    zeros = jnp.zeros((16,), jnp.float32)

    def zero_body(i, carry):
        zbuf[pl.ds(i * 16, 16)] = zeros
        return carry

    lax.fori_loop(0, 1024 // 16, zero_body, 0)

    @pl.when(s < 10)
    def _():
        off = pl.multiple_of(s * 1000, 8)
        pltpu.sync_copy(zbuf.at[pl.ds(0, 1000)],
                        deg_sh.at[pl.ds(off, 1000)])

    plsc.subcore_barrier()

    def fire(ci):
        off = pl.multiple_of(ci * CH, 8)
        pltpu.async_copy(w_v.at[pl.ds(off, CH)],
                         deg_sh.at[dst_v.at[ci]], sem, add=True)

    def wait_one():
        pltpu.make_async_copy(w_v.at[pl.ds(0, CH)],
                              deg_sh.at[dst_v.at[0]], sem).wait()

    fire(0)
    fire(1)
    fire(2)
    fire(3)

    def body(ci, carry):
        wait_one()
        fire(ci)
        return carry

    lax.fori_loop(4, NCHUNK, body, 0)
    for _ in range(4):
        wait_one()
    plsc.subcore_barrier()

    @pl.when(s < 10)
    def _():
        off = pl.multiple_of(s * 1000, 8)
        oof = pl.multiple_of(c * N + s * 1000, 8)
        pltpu.sync_copy(deg_sh.at[pl.ds(off, 1000)],
                        zbuf.at[pl.ds(0, 1000)])
        pltpu.sync_copy(zbuf.at[pl.ds(0, 1000)],
                        out_hbm.at[pl.ds(oof, 1000)])


def _sc_deg(dst3, w2):
    k = functools.partial(
        pl.kernel,
        mesh=_mesh,
        out_type=jax.ShapeDtypeStruct((NC * N,), jnp.float32),
        scratch_types=[
            pltpu.VMEM((NCHUNK, CH), jnp.int32),
            pltpu.VMEM((EPT,), jnp.float32),
            pltpu.VMEM((1024,), jnp.float32),
            pltpu.VMEM_SHARED((N,), jnp.float32),
            pltpu.SemaphoreType.DMA,
        ],
    )(_sc_deg_body)
    return k(dst3, w2)


def _sc_edge_body(src_hbm, dst_hbm, w_hbm, hs_hbm, out_hbm,
                  dst_v, src_v, w_v, rin0, rin1, rout0, rout1,
                  acc_sh, g0, g1, h0, h1, t0, t1, s0, s1):
    c = lax.axis_index("c")
    s = lax.axis_index("s")
    wid = s * NC + c
    rin = (rin0, rin1)
    rout = (rout0, rout1)
    gsem = (g0, g1)
    hsem = (h0, h1)
    tsem = (t0, t1)
    ssem = (s0, s1)

    zeros = jnp.zeros((16,), jnp.float32)
    for i in range(CH):
        for j in range(D // 16):
            rout0[i, pl.ds(j * 16, 16)] = zeros

    for k in range(8):
        zi = s + k * NS

        @pl.when(zi < NCHUNK)
        def _():
            off = pl.multiple_of(zi * CH, 8)
            pltpu.sync_copy(rout0, acc_sh.at[pl.ds(off, CH)])

    def fire_stage(ci, b2, b4):
        pltpu.async_copy(src_hbm.at[wid, ci], src_v.at[b2], tsem[b2])
        pltpu.async_copy(dst_hbm.at[wid, ci], dst_v.at[b4], tsem[b2])
        pltpu.async_copy(w_hbm.at[wid, ci], w_v.at[b2], tsem[b2])

    def wait_stage(b2):
        pltpu.make_async_copy(src_hbm.at[0, 0], src_v.at[b2], tsem[b2]).wait()
        pltpu.make_async_copy(src_hbm.at[0, 0], src_v.at[b2], tsem[b2]).wait()
        pltpu.make_async_copy(w_hbm.at[0, 0], w_v.at[b2], tsem[b2]).wait()

    H1 = 48
    H2 = CH - H1

    def fire_gather(b2):
        pltpu.async_copy(hs_hbm.at[src_v.at[b2, pl.ds(0, H1)]],
                         rin[b2].at[pl.ds(0, H1)], gsem[b2])
        pltpu.async_copy(hs_hbm.at[src_v.at[b2, pl.ds(H1, H2)]],
                         rin[b2].at[pl.ds(H1, H2)], hsem[b2])

    def wait_gather_half(b2, half):
        if half == 0:
            pltpu.make_async_copy(hs_hbm.at[src_v.at[b2, pl.ds(0, H1)]],
                                  rin[b2].at[pl.ds(0, H1)],
                                  gsem[b2]).wait()
        else:
            pltpu.make_async_copy(hs_hbm.at[src_v.at[b2, pl.ds(H1, H2)]],
                                  rin[b2].at[pl.ds(H1, H2)],
                                  hsem[b2]).wait()

    def wait_scatter(b2):
        pltpu.make_async_copy(rout[b2], acc_sh.at[dst_v.at[0]],
                              ssem[b2]).wait()

    def scale(b2, g_lo, g_hi):
        rv = rin[b2]
        ro = rout[b2]
        for g in range(g_lo, g_hi):
            w_vec = w_v[b2, pl.ds(g * 16, 16)]
            for r in range(16):
                e = g * 16 + r
                ws = _splat(w_vec, r)
                for j in range(D // 16):
                    ro[e, pl.ds(j * 16, 16)] = (
                        rv[e, pl.ds(j * 16, 16)] * ws)

    def phase(ci, b2, b4):
        bp2 = (b2 + 1) % 2

        @pl.when(ci + 1 < NCHUNK)
        def _():
            wait_stage(bp2)
            fire_gather(bp2)

        wait_gather_half(b2, 0)

        @pl.when(ci >= 2)
        def _():
            wait_scatter(b2)

        scale(b2, 0, H1 // 16)
        wait_gather_half(b2, 1)
        scale(b2, H1 // 16, CH // 16)
        pltpu.async_copy(rout[b2], acc_sh.at[dst_v.at[b4]],
                         ssem[b2], add=True)

        @pl.when(ci + 2 < NCHUNK)
        def _():
            fire_stage(ci + 2, b2, (b4 + 2) % 4)

    fire_stage(0, 0, 0)
    fire_stage(1, 1, 1)
    plsc.subcore_barrier()
    wait_stage(0)
    fire_gather(0)

    def pair(k, carry):
        ci = k * 2
        b4a = (k % 2) * 2
        phase(ci, 0, b4a)
        phase(ci + 1, 1, b4a + 1)
        return carry

    lax.fori_loop(0, NCHUNK // 2, pair, 0)
    phase(NCHUNK - 1, 0, 0)
    wait_scatter(1)
    wait_scatter(0)
    plsc.subcore_barrier()

    for k in range(8):
        ci = s + k * NS

        @pl.when(ci < NCHUNK)
        def _():
            off = pl.multiple_of(ci * CH, 8)
            oof = pl.multiple_of(c * N + ci * CH, 8)
            pltpu.sync_copy(acc_sh.at[pl.ds(off, CH)], rout0)
            pltpu.sync_copy(rout0, out_hbm.at[pl.ds(oof, CH)])


def _sc_edges(src2, dst3, w2, hs):
    k = functools.partial(
        pl.kernel,
        mesh=_mesh,
        out_type=jax.ShapeDtypeStruct((NC * N, D), jnp.float32),
        scratch_types=[
            pltpu.VMEM((4, CH), jnp.int32),
            pltpu.VMEM((2, CH), jnp.int32),
            pltpu.VMEM((2, CH), jnp.float32),
            pltpu.VMEM((CH, D), jnp.float32),
            pltpu.VMEM((CH, D), jnp.float32),
            pltpu.VMEM((CH, D), jnp.float32),
            pltpu.VMEM((CH, D), jnp.float32),
            pltpu.VMEM_SHARED((N, D), jnp.float32),
        ] + [pltpu.SemaphoreType.DMA] * 8,
    )(_sc_edge_body)
    return k(src2, dst3, w2, hs)


_BLK = 1000


def _tc_mm_body(x_ref, w_ref, degp_ref, h_ref, hs_ref):
    xb = x_ref[...]
    h = jnp.dot(xb, w_ref[...], preferred_element_type=jnp.float32)
    deg = 1.0 + jnp.sum(degp_ref[...], axis=1, keepdims=True)
    dinv = lax.rsqrt(deg)
    h_ref[...] = h
    hs_ref[...] = h * dinv


def _tc_mm(x, W, degp_t):
    grid = (N // _BLK,)
    return pl.pallas_call(
        _tc_mm_body,
        grid=grid,
        in_specs=[
            pl.BlockSpec((_BLK, D), lambda i: (i, 0)),
            pl.BlockSpec((D, D), lambda i: (0, 0)),
            pl.BlockSpec((_BLK, NC), lambda i: (i, 0)),
        ],
        out_specs=[
            pl.BlockSpec((_BLK, D), lambda i: (i, 0)),
            pl.BlockSpec((_BLK, D), lambda i: (i, 0)),
        ],
        out_shape=[
            jax.ShapeDtypeStruct((N, D), jnp.float32),
            jax.ShapeDtypeStruct((N, D), jnp.float32),
        ],
    )(x, W, degp_t)


def _tc_final_body(acc_ref, h_ref, degp_ref, b_ref, g_ref, be_ref, o_ref):
    acc = acc_ref[0] + acc_ref[1]
    deg = 1.0 + jnp.sum(degp_ref[...], axis=1, keepdims=True)
    dinv = lax.rsqrt(deg)
    pre = acc * dinv + h_ref[...] * (dinv * dinv) + b_ref[...]
    mean = jnp.mean(pre, axis=0, keepdims=True)
    var = jnp.mean((pre - mean) * (pre - mean), axis=0, keepdims=True)
    o = (pre - mean) * lax.rsqrt(var + 1e-5) * g_ref[...] + be_ref[...]
    o_ref[...] = jnp.maximum(o, 0.0)


def _tc_final(acc, h, degp_t, b, gamma, beta):
    return pl.pallas_call(
        _tc_final_body,
        out_shape=jax.ShapeDtypeStruct((N, D), jnp.float32),
    )(acc, h, degp_t, b, gamma, beta)


def kernel(x, edge_index, edge_weight, W, b, gamma, beta):
    src = edge_index[0]
    dst = edge_index[1]
    src3 = src.reshape(NW, NCHUNK, CH)
    dst3 = dst.reshape(NW, NCHUNK, CH)
    w3 = edge_weight.reshape(NW, NCHUNK, CH)
    w2 = edge_weight.reshape(NW, 1, EPT)

    degp = _sc_deg(dst3, w2).reshape(NC, N)
    degp_t = degp.T
    h, hs = _tc_mm(x, W, degp_t)
    acc = _sc_edges(src3, dst3, w3, hs).reshape(NC, N, D)
    out = _tc_final(acc, h, degp_t,
                    b.reshape(1, D), gamma.reshape(1, D), beta.reshape(1, D))
    return out
